# Initial kernel scaffold; baseline (speedup 1.0000x reference)
import jax
import jax.numpy as jnp
from jax.experimental import pallas as pl


def kernel(pos, normal, batch, w1a, b1a, w2a, b2a, w1b, b1b, w2b, b2b, wc, bc):
    raise NotImplementedError("write your pallas kernel here")


# speedup vs baseline: 9.1443x; 9.1443x over previous
import functools

import jax
import jax.numpy as jnp
import numpy as np
from jax import lax
from jax.experimental import pallas as pl
from jax.experimental.pallas import tpu as pltpu
from jax.experimental.pallas import tpu_sc as plsc

N = 10000
K = 16
NUM_GRAPHS = 64
NUM_CLASSES = 40

R = 256
NP = 10240
NT = NP // R
C = 256
EP = NP * K
ET = R * K
BIGV = 1e30
BIGI = 2**30


def _knn_body(pos_ref, post_ref, batch_ref, bcol_ref, out_ref):
    i = pl.program_id(0)
    pos_r = pos_ref[pl.ds(i * R, R), :]
    sq_r = jnp.sum(pos_r * pos_r, axis=1, keepdims=True)
    b_rc = bcol_ref[pl.ds(i * R, R), :]
    row_ids = i * R + lax.broadcasted_iota(jnp.int32, (R, 1), 0)

    b_all = batch_ref[...]

    b_lo = jnp.min(b_rc)
    b_hi = jnp.max(b_rc)
    colio = lax.broadcasted_iota(jnp.int32, (1, NP), 1)
    span = (b_all >= b_lo) & (b_all <= b_hi)
    lo = jnp.min(jnp.where(span, colio, NP))
    hi = jnp.max(jnp.where(span, colio, -1))
    c_lo = lo // C
    c_hi = hi // C

    def merge(c, carry):
        bv, bi = carry
        pos_c = post_ref[:, pl.ds(c * C, C)]
        sq_c = jnp.sum(pos_c * pos_c, axis=0, keepdims=True)
        b_c = batch_ref[0:1, pl.ds(c * C, C)]
        dist = sq_r + sq_c - 2.0 * jnp.dot(
            pos_r, pos_c, preferred_element_type=jnp.float32)
        dist = jnp.maximum(dist, 0.0)
        col_ids = c * C + lax.broadcasted_iota(jnp.int32, (1, C), 1)
        dist = jnp.where(b_rc != b_c, 1e9, dist)
        dist = jnp.where(col_ids == row_ids, 1e9, dist)
        cv = jnp.concatenate([bv, dist], axis=1)
        ci = jnp.concatenate([bi, jnp.broadcast_to(col_ids, (R, C))], axis=1)
        vs, js = [], []
        for _ in range(K):
            m = jnp.min(cv, axis=1, keepdims=True)
            sel = jnp.min(jnp.where(cv == m, ci, BIGI), axis=1, keepdims=True)
            vs.append(m)
            js.append(sel)
            cv = jnp.where(ci == sel, BIGV, cv)
        return jnp.concatenate(vs, axis=1), jnp.concatenate(js, axis=1)

    bv0 = jnp.full((R, K), BIGV, jnp.float32)
    bi0 = jnp.full((R, K), BIGI, jnp.int32)
    bv, bi = merge(0, (bv0, bi0))
    bv, bi = lax.fori_loop(jnp.maximum(c_lo, 1), c_hi + 1, merge, (bv, bi))
    out_ref[0] = bi


def _knn(posp, postp, batchp, bcolp):
    return pl.pallas_call(
        _knn_body,
        grid=(NT,),
        in_specs=[
            pl.BlockSpec((NP, 8), lambda i: (0, 0)),
            pl.BlockSpec((8, NP), lambda i: (0, 0)),
            pl.BlockSpec((1, NP), lambda i: (0, 0)),
            pl.BlockSpec((NP, 1), lambda i: (0, 0)),
        ],
        out_specs=pl.BlockSpec((1, R, K), lambda i: (i, 0, 0)),
        out_shape=jax.ShapeDtypeStruct((NT, R, K), jnp.int32),
    )(posp, postp, batchp, bcolp)


def _sc_gather(table, idx, chunk):
    v, d = table.shape
    b = idx.shape[0]
    info = plsc.get_sparse_core_info()
    nw = info.num_cores * info.num_subcores
    b_per_w = b // nw
    nchunks = b_per_w // chunk
    mesh = plsc.VectorSubcoreMesh(core_axis_name="c", subcore_axis_name="s")

    @functools.partial(
        pl.kernel,
        mesh=mesh,
        compiler_params=pltpu.CompilerParams(use_tc_tiling_on_sc=False),
        out_type=jax.ShapeDtypeStruct((b, d), jnp.float32),
        scratch_types=[
            pltpu.VMEM((chunk,), jnp.int32),
            pltpu.VMEM((chunk, d), jnp.float32),
            pltpu.SemaphoreType.DMA,
        ],
    )
    def k(table_hbm, idx_hbm, out_hbm, idx_v, rows_v, sem):
        wid = lax.axis_index("s") * info.num_cores + lax.axis_index("c")
        base = wid * b_per_w
        for j in range(nchunks):
            off = base + j * chunk
            pltpu.sync_copy(idx_hbm.at[pl.ds(off, chunk)], idx_v)
            pltpu.async_copy(table_hbm.at[idx_v], rows_v, sem).wait()
            pltpu.sync_copy(rows_v, out_hbm.at[pl.ds(off, chunk)])

    return k(table, idx)


def _selmat(src_cols, dst_cols):
    m = np.zeros((16, 16), np.float32)
    for s, t in zip(src_cols, dst_cols):
        m[s, t] = 1.0
    return m

_SEL_NRM_NP = _selmat((3, 4, 5), (0, 1, 2))
_ROLL1_NP = _selmat((1, 2, 0), (0, 1, 2))
_ROLL2_NP = _selmat((2, 0, 1), (0, 1, 2))
_SEL_POS_NP = _selmat((0, 1, 2), (0, 1, 2))


def _dot3(a, b):
    return jnp.sum(a * b, axis=1, keepdims=True)


def _conv1_body(g_ref, pn_ref, sel_ref, w1_ref, b1_ref, w2_ref, b2_ref,
                x_ref, f_ref):
    selp = sel_ref[0]
    seln = sel_ref[1]
    roll1 = sel_ref[2]
    roll2 = sel_ref[3]

    def mm(a, b):
        return jnp.dot(a, b, preferred_element_type=jnp.float32)

    def crossnorm(r1a, r2a, b):
        cr = r1a * mm(b, roll2) - r2a * mm(b, roll1)
        return jnp.sqrt(_dot3(cr, cr))

    g = g_ref[...]
    pn = pn_ref[...]
    pne = jnp.broadcast_to(pn[:, None, :], (R, K, 16)).reshape(ET, 16)
    pj = mm(g, selp)
    nj = mm(g, seln)
    pi = mm(pne, selp)
    ni = mm(pne, seln)
    d = pj - pi
    nd = jnp.sqrt(_dot3(d, d))
    r1ni = mm(ni, roll1)
    r2ni = mm(ni, roll2)
    r1nj = mm(nj, roll1)
    r2nj = mm(nj, roll2)
    a1 = jnp.arctan2(crossnorm(r1ni, r2ni, d), _dot3(ni, d))
    a2 = jnp.arctan2(crossnorm(r1nj, r2nj, d), _dot3(nj, d))
    a3 = jnp.arctan2(crossnorm(r1ni, r2ni, nj), _dot3(ni, nj))
    f = jnp.concatenate([nd, a1, a2, a3], axis=1)
    f_ref[...] = f
    h = jnp.dot(jax.nn.relu(
        jnp.dot(f, w1_ref[...], preferred_element_type=jnp.float32)
        + b1_ref[...]), w2_ref[...],
        preferred_element_type=jnp.float32) + b2_ref[...]
    x = jnp.max(h.reshape(R, K, 32), axis=1)
    x_ref[...] = jax.nn.relu(x)


def _conv1(g, pnp, sel, w1a, b1a, w2a, b2a):
    return pl.pallas_call(
        _conv1_body,
        grid=(NT,),
        in_specs=[
            pl.BlockSpec((ET, 16), lambda i: (i, 0)),
            pl.BlockSpec((R, 16), lambda i: (i, 0)),
            pl.BlockSpec((4, 16, 16), lambda i: (0, 0, 0)),
            pl.BlockSpec((4, 32), lambda i: (0, 0)),
            pl.BlockSpec((1, 32), lambda i: (0, 0)),
            pl.BlockSpec((32, 32), lambda i: (0, 0)),
            pl.BlockSpec((1, 32), lambda i: (0, 0)),
        ],
        out_specs=[
            pl.BlockSpec((R, 32), lambda i: (i, 0)),
            pl.BlockSpec((ET, 4), lambda i: (i, 0)),
        ],
        out_shape=[
            jax.ShapeDtypeStruct((NP, 32), jnp.float32),
            jax.ShapeDtypeStruct((EP, 4), jnp.float32),
        ],
    )(g, pnp, sel, w1a, b1a, w2a, b2a)


def _conv2_body(xg_ref, f_ref, bcol_ref, w1_ref, b1_ref, w2_ref, b2_ref,
                wc_ref, bc_ref, out_ref, acc_ref):
    i = pl.program_id(0)
    msg = jnp.concatenate([xg_ref[...], f_ref[...]], axis=1)
    h = jnp.dot(jax.nn.relu(
        jnp.dot(msg, w1_ref[...], preferred_element_type=jnp.float32)
        + b1_ref[...]), w2_ref[...],
        preferred_element_type=jnp.float32) + b2_ref[...]
    x2 = jax.nn.relu(jnp.max(h.reshape(R, K, 32), axis=1))
    bcol = bcol_ref[...]
    pooled = jnp.concatenate(
        [jnp.max(jnp.where(bcol == g, x2, -jnp.inf), axis=0, keepdims=True)
         for g in range(NUM_GRAPHS)], axis=0)

    @pl.when(i == 0)
    def _():
        acc_ref[...] = jnp.full((NUM_GRAPHS, 32), -jnp.inf, jnp.float32)

    acc_ref[...] = jnp.maximum(acc_ref[...], pooled)

    @pl.when(i == NT - 1)
    def _():
        out_ref[...] = jnp.dot(
            acc_ref[...], wc_ref[...],
            preferred_element_type=jnp.float32) + bc_ref[...]


def _conv2(xg, f, bcolp, w1b, b1b, w2b, b2b, wc, bc):
    return pl.pallas_call(
        _conv2_body,
        grid=(NT,),
        in_specs=[
            pl.BlockSpec((ET, 32), lambda i: (i, 0)),
            pl.BlockSpec((ET, 4), lambda i: (i, 0)),
            pl.BlockSpec((R, 1), lambda i: (i, 0)),
            pl.BlockSpec((36, 32), lambda i: (0, 0)),
            pl.BlockSpec((1, 32), lambda i: (0, 0)),
            pl.BlockSpec((32, 32), lambda i: (0, 0)),
            pl.BlockSpec((1, 32), lambda i: (0, 0)),
            pl.BlockSpec((32, NUM_CLASSES), lambda i: (0, 0)),
            pl.BlockSpec((1, NUM_CLASSES), lambda i: (0, 0)),
        ],
        out_specs=pl.BlockSpec((NUM_GRAPHS, NUM_CLASSES), lambda i: (0, 0)),
        out_shape=jax.ShapeDtypeStruct((NUM_GRAPHS, NUM_CLASSES), jnp.float32),
        scratch_shapes=[pltpu.VMEM((NUM_GRAPHS, 32), jnp.float32)],
    )(xg, f, bcolp, w1b, b1b, w2b, b2b, wc, bc)


def kernel(pos, normal, batch, w1a, b1a, w2a, b2a, w1b, b1b, w2b, b2b, wc, bc):
    b32 = batch.astype(jnp.int32)
    posp = jnp.zeros((NP, 8), jnp.float32).at[:N, :3].set(pos)
    postp = posp.T
    batchp = jnp.full((1, NP), 127, jnp.int32).at[0, :N].set(b32)
    bcolp = batchp.reshape(NP, 1)
    pnp = jnp.zeros((NP, 16), jnp.float32)
    pnp = pnp.at[:N, 0:3].set(pos).at[:N, 3:6].set(normal)

    nbr = _knn(posp, postp, batchp, bcolp)
    src = jnp.clip(nbr.reshape(EP), 0, NP - 1)

    sel = jnp.asarray(
        np.stack([_SEL_POS_NP, _SEL_NRM_NP, _ROLL1_NP, _ROLL2_NP]))

    g = _sc_gather(pnp, src, chunk=5120)
    x, f = _conv1(g, pnp, sel, w1a, b1a.reshape(1, 32), w2a,
                  b2a.reshape(1, 32))
    xg = _sc_gather(x, src, chunk=2560)
    return _conv2(xg, f, bcolp, w1b, b1b.reshape(1, 32), w2b,
                  b2b.reshape(1, 32), wc, bc.reshape(1, NUM_CLASSES))


# speedup vs baseline: 9.9611x; 1.0893x over previous
import functools

import jax
import jax.numpy as jnp
import numpy as np
from jax import lax
from jax.experimental import pallas as pl
from jax.experimental.pallas import tpu as pltpu
from jax.experimental.pallas import tpu_sc as plsc

N = 10000
K = 16
NUM_GRAPHS = 64
NUM_CLASSES = 40

R = 256
NP = 10240
NT = NP // R
C = 256
EP = NP * K
ET = R * K
BIGV = 1e30
BIGI = 2**30


def _knn_body(bounds_ref, pos_ref, post_ref, batch_ref, bcol_ref, out_ref):
    i = pl.program_id(0)
    pos_r = pos_ref[pl.ds(i * R, R), :]
    sq_r = jnp.sum(pos_r * pos_r, axis=1, keepdims=True)
    b_rc = bcol_ref[pl.ds(i * R, R), :]
    row_ids = i * R + lax.broadcasted_iota(jnp.int32, (R, 1), 0)

    c_lo = bounds_ref[i, 0]
    c_hi = bounds_ref[i, 1]

    def merge(c, carry):
        bv, bi = carry
        pos_c = post_ref[:, pl.ds(c * C, C)]
        sq_c = jnp.sum(pos_c * pos_c, axis=0, keepdims=True)
        b_c = batch_ref[0:1, pl.ds(c * C, C)]
        dist = sq_r + sq_c - 2.0 * jnp.dot(
            pos_r, pos_c, preferred_element_type=jnp.float32)
        dist = jnp.maximum(dist, 0.0)
        col_ids = c * C + lax.broadcasted_iota(jnp.int32, (1, C), 1)
        dist = jnp.where(b_rc != b_c, 1e9, dist)
        dist = jnp.where(col_ids == row_ids, 1e9, dist)
        cv = jnp.concatenate([bv, dist], axis=1)
        ci = jnp.concatenate([bi, jnp.broadcast_to(col_ids, (R, C))], axis=1)
        vs, js = [], []
        for _ in range(K):
            m = jnp.min(cv, axis=1, keepdims=True)
            sel = jnp.min(jnp.where(cv == m, ci, BIGI), axis=1, keepdims=True)
            vs.append(m)
            js.append(sel)
            cv = jnp.where(ci == sel, BIGV, cv)
        return jnp.concatenate(vs, axis=1), jnp.concatenate(js, axis=1)

    bv0 = jnp.full((R, K), BIGV, jnp.float32)
    bi0 = jnp.full((R, K), BIGI, jnp.int32)
    bv, bi = lax.fori_loop(c_lo, c_hi + 1, merge, (bv0, bi0))
    need0 = (c_lo > 0) & (jnp.max(bv[:, K - 1:K]) >= 1e9)
    bv, bi = lax.cond(need0, lambda a, b: merge(0, (a, b)),
                      lambda a, b: (a, b), bv, bi)
    out_ref[0] = bi


def _knn(bounds, posp, postp, batchp, bcolp):
    return pl.pallas_call(
        _knn_body,
        grid=(NT,),
        in_specs=[
            pl.BlockSpec(memory_space=pltpu.SMEM),
            pl.BlockSpec((NP, 8), lambda i: (0, 0)),
            pl.BlockSpec((8, NP), lambda i: (0, 0)),
            pl.BlockSpec((1, NP), lambda i: (0, 0)),
            pl.BlockSpec((NP, 1), lambda i: (0, 0)),
        ],
        out_specs=pl.BlockSpec((1, R, K), lambda i: (i, 0, 0)),
        out_shape=jax.ShapeDtypeStruct((NT, R, K), jnp.int32),
    )(bounds, posp, postp, batchp, bcolp)


def _sc_gather(table, idx, chunk):
    v, d = table.shape
    b = idx.shape[0]
    info = plsc.get_sparse_core_info()
    nw = info.num_cores * info.num_subcores
    b_per_w = b // nw
    nchunks = b_per_w // chunk
    mesh = plsc.VectorSubcoreMesh(core_axis_name="c", subcore_axis_name="s")

    @functools.partial(
        pl.kernel,
        mesh=mesh,
        compiler_params=pltpu.CompilerParams(use_tc_tiling_on_sc=False),
        out_type=jax.ShapeDtypeStruct((b, d), jnp.float32),
        scratch_types=[
            pltpu.VMEM((chunk,), jnp.int32),
            pltpu.VMEM((chunk, d), jnp.float32),
            pltpu.SemaphoreType.DMA,
        ],
    )
    def k(table_hbm, idx_hbm, out_hbm, idx_v, rows_v, sem):
        wid = lax.axis_index("s") * info.num_cores + lax.axis_index("c")
        base = wid * b_per_w
        for j in range(nchunks):
            off = base + j * chunk
            pltpu.sync_copy(idx_hbm.at[pl.ds(off, chunk)], idx_v)
            pltpu.async_copy(table_hbm.at[idx_v], rows_v, sem).wait()
            pltpu.sync_copy(rows_v, out_hbm.at[pl.ds(off, chunk)])

    return k(table, idx)


def _selmat(src_cols, dst_cols):
    m = np.zeros((16, 16), np.float32)
    for s, t in zip(src_cols, dst_cols):
        m[s, t] = 1.0
    return m

_SEL_NRM_NP = _selmat((3, 4, 5), (0, 1, 2))
_ROLL1_NP = _selmat((1, 2, 0), (0, 1, 2))
_ROLL2_NP = _selmat((2, 0, 1), (0, 1, 2))
_SEL_POS_NP = _selmat((0, 1, 2), (0, 1, 2))


def _dot3(a, b):
    return jnp.sum(a * b, axis=1, keepdims=True)


def _conv1_body(g_ref, pn_ref, sel_ref, w1_ref, b1_ref, w2_ref, b2_ref,
                x_ref, f_ref):
    selp = sel_ref[0]
    seln = sel_ref[1]
    roll1 = sel_ref[2]
    roll2 = sel_ref[3]

    def mm(a, b):
        return jnp.dot(a, b, preferred_element_type=jnp.float32)

    def crossnorm(r1a, r2a, b):
        cr = r1a * mm(b, roll2) - r2a * mm(b, roll1)
        return jnp.sqrt(_dot3(cr, cr))

    g = g_ref[...]
    pn = pn_ref[...]
    pne = jnp.broadcast_to(pn[:, None, :], (R, K, 16)).reshape(ET, 16)
    pj = mm(g, selp)
    nj = mm(g, seln)
    pi = mm(pne, selp)
    ni = mm(pne, seln)
    d = pj - pi
    nd = jnp.sqrt(_dot3(d, d))
    r1ni = mm(ni, roll1)
    r2ni = mm(ni, roll2)
    r1nj = mm(nj, roll1)
    r2nj = mm(nj, roll2)
    a1 = jnp.arctan2(crossnorm(r1ni, r2ni, d), _dot3(ni, d))
    a2 = jnp.arctan2(crossnorm(r1nj, r2nj, d), _dot3(nj, d))
    a3 = jnp.arctan2(crossnorm(r1ni, r2ni, nj), _dot3(ni, nj))
    f = jnp.concatenate([nd, a1, a2, a3], axis=1)
    f_ref[...] = f
    h = jnp.dot(jax.nn.relu(
        jnp.dot(f, w1_ref[...], preferred_element_type=jnp.float32)
        + b1_ref[...]), w2_ref[...],
        preferred_element_type=jnp.float32) + b2_ref[...]
    x = jnp.max(h.reshape(R, K, 32), axis=1)
    x_ref[...] = jax.nn.relu(x)


def _conv1(g, pnp, sel, w1a, b1a, w2a, b2a):
    return pl.pallas_call(
        _conv1_body,
        grid=(NT,),
        in_specs=[
            pl.BlockSpec((ET, 16), lambda i: (i, 0)),
            pl.BlockSpec((R, 16), lambda i: (i, 0)),
            pl.BlockSpec((4, 16, 16), lambda i: (0, 0, 0)),
            pl.BlockSpec((4, 32), lambda i: (0, 0)),
            pl.BlockSpec((1, 32), lambda i: (0, 0)),
            pl.BlockSpec((32, 32), lambda i: (0, 0)),
            pl.BlockSpec((1, 32), lambda i: (0, 0)),
        ],
        out_specs=[
            pl.BlockSpec((R, 32), lambda i: (i, 0)),
            pl.BlockSpec((ET, 4), lambda i: (i, 0)),
        ],
        out_shape=[
            jax.ShapeDtypeStruct((NP, 32), jnp.float32),
            jax.ShapeDtypeStruct((EP, 4), jnp.float32),
        ],
    )(g, pnp, sel, w1a, b1a, w2a, b2a)


def _conv2_body(xg_ref, f_ref, bcol_ref, w1_ref, b1_ref, w2_ref, b2_ref,
                wc_ref, bc_ref, out_ref, acc_ref):
    i = pl.program_id(0)
    msg = jnp.concatenate([xg_ref[...], f_ref[...]], axis=1)
    h = jnp.dot(jax.nn.relu(
        jnp.dot(msg, w1_ref[...], preferred_element_type=jnp.float32)
        + b1_ref[...]), w2_ref[...],
        preferred_element_type=jnp.float32) + b2_ref[...]
    x2 = jax.nn.relu(jnp.max(h.reshape(R, K, 32), axis=1))
    bcol = bcol_ref[...]
    pooled = jnp.concatenate(
        [jnp.max(jnp.where(bcol == g, x2, -jnp.inf), axis=0, keepdims=True)
         for g in range(NUM_GRAPHS)], axis=0)

    @pl.when(i == 0)
    def _():
        acc_ref[...] = jnp.full((NUM_GRAPHS, 32), -jnp.inf, jnp.float32)

    acc_ref[...] = jnp.maximum(acc_ref[...], pooled)

    @pl.when(i == NT - 1)
    def _():
        out_ref[...] = jnp.dot(
            acc_ref[...], wc_ref[...],
            preferred_element_type=jnp.float32) + bc_ref[...]


def _conv2(xg, f, bcolp, w1b, b1b, w2b, b2b, wc, bc):
    return pl.pallas_call(
        _conv2_body,
        grid=(NT,),
        in_specs=[
            pl.BlockSpec((ET, 32), lambda i: (i, 0)),
            pl.BlockSpec((ET, 4), lambda i: (i, 0)),
            pl.BlockSpec((R, 1), lambda i: (i, 0)),
            pl.BlockSpec((36, 32), lambda i: (0, 0)),
            pl.BlockSpec((1, 32), lambda i: (0, 0)),
            pl.BlockSpec((32, 32), lambda i: (0, 0)),
            pl.BlockSpec((1, 32), lambda i: (0, 0)),
            pl.BlockSpec((32, NUM_CLASSES), lambda i: (0, 0)),
            pl.BlockSpec((1, NUM_CLASSES), lambda i: (0, 0)),
        ],
        out_specs=pl.BlockSpec((NUM_GRAPHS, NUM_CLASSES), lambda i: (0, 0)),
        out_shape=jax.ShapeDtypeStruct((NUM_GRAPHS, NUM_CLASSES), jnp.float32),
        scratch_shapes=[pltpu.VMEM((NUM_GRAPHS, 32), jnp.float32)],
    )(xg, f, bcolp, w1b, b1b, w2b, b2b, wc, bc)


def kernel(pos, normal, batch, w1a, b1a, w2a, b2a, w1b, b1b, w2b, b2b, wc, bc):
    b32 = batch.astype(jnp.int32)
    posp = jnp.zeros((NP, 8), jnp.float32).at[:N, :3].set(pos)
    postp = posp.T
    batchp = jnp.full((1, NP), 127, jnp.int32).at[0, :N].set(b32)
    bcolp = batchp.reshape(NP, 1)
    pnp = jnp.zeros((NP, 16), jnp.float32)
    pnp = pnp.at[:N, 0:3].set(pos).at[:N, 3:6].set(normal)

    bp = batchp[0]
    tiles = jnp.arange(NT)
    lo_v = jnp.searchsorted(bp, bp[tiles * R], side="left")
    hi_v = jnp.searchsorted(bp, bp[tiles * R + R - 1], side="right") - 1
    bounds = jnp.stack([lo_v // C, hi_v // C], axis=1).astype(jnp.int32)

    nbr = _knn(bounds, posp, postp, batchp, bcolp)
    src = jnp.clip(nbr.reshape(EP), 0, NP - 1)

    sel = jnp.asarray(
        np.stack([_SEL_POS_NP, _SEL_NRM_NP, _ROLL1_NP, _ROLL2_NP]))

    g = _sc_gather(pnp, src, chunk=5120)
    x, f = _conv1(g, pnp, sel, w1a, b1a.reshape(1, 32), w2a,
                  b2a.reshape(1, 32))
    xg = _sc_gather(x, src, chunk=2560)
    return _conv2(xg, f, bcolp, w1b, b1b.reshape(1, 32), w2b,
                  b2b.reshape(1, 32), wc, bc.reshape(1, NUM_CLASSES))
